# baseline (speedup 1.0000x reference)
import functools

import jax
import jax.numpy as jnp
from jax.experimental import pallas as pl
from jax.experimental.pallas import tpu as pltpu

_MAX_SPAN_LEN = 16
_WIN = 24


def _pool_body(pk_sm, n1_sm, n2_sm, wr_sm, sn_sm, emb_hbm, sv_ref, ev_ref,
               out_ref, mask_ref, gbuf, sem, *, S, D):
    b = pl.program_id(0)
    nb = pl.num_programs(0)
    slot = jax.lax.rem(b, 4)

    def issue(bb, sl):
        def cp(j, rows):
            v = pk_sm[bb * S + j]
            src = pl.multiple_of(v & 0xFFFF, 8)
            dst = pl.multiple_of((v >> 16) * _WIN, 8)
            pltpu.make_async_copy(
                emb_hbm.at[bb, pl.ds(src, rows), :],
                gbuf.at[sl, pl.ds(dst, rows), :],
                sem.at[sl],
            ).start()

        n1 = n1_sm[bb]
        n12 = n1 + n2_sm[bb]
        jax.lax.fori_loop(0, n1, lambda j, c: (cp(j, 8), c)[1], 0)
        jax.lax.fori_loop(n1, n12, lambda j, c: (cp(j, 16), c)[1], 0)
        jax.lax.fori_loop(n12, S, lambda j, c: (cp(j, 24), c)[1], 0)

    @pl.when(b == 0)
    def _():
        issue(0, 0)
        issue(1, 1)

    @pl.when(b + 2 < nb)
    def _():
        issue(b + 2, jax.lax.rem(b + 2, 4))

    sn = sn_sm[b]
    span_iota = jax.lax.broadcasted_iota(jnp.int32, (1, S), 1)
    valid_span = span_iota < sn
    mask_ref[0] = valid_span.astype(jnp.int32)

    sv = sv_ref[0]
    ev = ev_ref[0]
    lo = sv - ((sv >> 3) << 3)
    cnt = jnp.minimum(ev - sv, _MAX_SPAN_LEN)
    inv = 1.0 / jnp.maximum(cnt, 1).astype(jnp.float32)
    scale = jnp.where(valid_span & (cnt > 0), inv, 0.0)

    k_iota = jax.lax.broadcasted_iota(jnp.int32, (S * _WIN, S), 0)
    off = k_iota - (span_iota * _WIN + lo)
    in_span = off.astype(jnp.uint32) < cnt.astype(jnp.uint32)
    wt = jnp.where(in_span, scale, 0.0)

    wr = pl.multiple_of(wr_sm[b], 8)
    pltpu.make_async_copy(
        emb_hbm.at[b, pl.ds(0, wr), :],
        gbuf.at[slot, pl.ds(0, wr), :],
        sem.at[slot],
    ).wait()

    out_ref[0] = jax.lax.dot_general(
        wt, gbuf[slot], (((0,), (0,)), ((), ())),
        preferred_element_type=jnp.float32)


def kernel(embeddings, span_positions, span_nums, repeated_vector):
    B, L, D = embeddings.shape
    S = span_positions.shape[1]
    sp = span_positions.astype(jnp.int32) + 1
    starts = sp[..., 0]
    ends = sp[..., 1]
    lo = starts & 7
    cnt = jnp.clip(ends - starts, 0, _MAX_SPAN_LEN)
    base_rows = (starts >> 3) * 8
    ntile = (lo + jnp.maximum(cnt, 1) + 7) >> 3
    ntile = jnp.where(jnp.arange(B, dtype=jnp.int32)[:, None] <= 3, 3, ntile)

    order = jnp.argsort(ntile, axis=1, stable=True)
    base_sorted = jnp.take_along_axis(base_rows, order, axis=1)
    pk = (base_sorted | (order << 16)).reshape(-1)
    n1 = jnp.sum((ntile == 1).astype(jnp.int32), axis=1)
    n2 = jnp.sum((ntile == 2).astype(jnp.int32), axis=1)
    wr = 8 * jnp.sum(ntile, axis=1)

    sn = span_nums.astype(jnp.int32)
    sv = starts.reshape(B, 1, S)
    ev = ends.reshape(B, 1, S)

    body = functools.partial(_pool_body, S=S, D=D)
    grid_spec = pltpu.PrefetchScalarGridSpec(
        num_scalar_prefetch=5,
        grid=(B,),
        in_specs=[pl.BlockSpec(memory_space=pl.ANY),
                  pl.BlockSpec((1, 1, S), lambda b, *_: (b, 0, 0)),
                  pl.BlockSpec((1, 1, S), lambda b, *_: (b, 0, 0))],
        out_specs=[pl.BlockSpec((1, S, D), lambda b, *_: (b, 0, 0)),
                   pl.BlockSpec((1, 1, S), lambda b, *_: (b, 0, 0))],
        scratch_shapes=[
            pltpu.VMEM((4, S * _WIN, D), jnp.float32),
            pltpu.SemaphoreType.DMA((4,)),
        ],
    )
    pooled, maski = pl.pallas_call(
        body,
        grid_spec=grid_spec,
        out_shape=[jax.ShapeDtypeStruct((B, S, D), jnp.float32),
                   jax.ShapeDtypeStruct((B, 1, S), jnp.int32)],
        compiler_params=pltpu.CompilerParams(
            dimension_semantics=("arbitrary",),
        ),
        name="span_mean_pool_dma_mxu",
    )(pk, n1, n2, wr, sn, embeddings, sv, ev)
    return pooled, maski.reshape(B, S) > 0


# speedup vs baseline: 1.2042x; 1.2042x over previous
import functools

import jax
import jax.numpy as jnp
from jax.experimental import pallas as pl
from jax.experimental.pallas import tpu as pltpu

_MAX_SPAN_LEN = 16
_WIN = 24


def _pool_body(pk_sm, n1_sm, n2_sm, wr_sm, sn_sm, emb_hbm, sv_ref, ev_ref,
               out_ref, mask_ref, gbuf, sem, *, S, D):
    b = pl.program_id(0)
    nb = pl.num_programs(0)
    slot = jax.lax.rem(b, 4)

    def issue(bb, sl):
        def cp(j, rows):
            v = pk_sm[bb * S + j]
            src = pl.multiple_of(v & 0xFFFF, 8)
            dst = pl.multiple_of((v >> 16) * _WIN, 8)
            pltpu.make_async_copy(
                emb_hbm.at[bb, pl.ds(src, rows), :],
                gbuf.at[sl, pl.ds(dst, rows), :],
                sem.at[sl],
            ).start()

        n1 = n1_sm[bb]
        n12 = n1 + n2_sm[bb]
        jax.lax.fori_loop(0, n1, lambda j, c: (cp(j, 8), c)[1], 0)
        jax.lax.fori_loop(n1, n12, lambda j, c: (cp(j, 16), c)[1], 0)
        jax.lax.fori_loop(n12, S, lambda j, c: (cp(j, 24), c)[1], 0)

    @pl.when(b == 0)
    def _():
        issue(0, 0)
        issue(1, 1)

    @pl.when(b + 2 < nb)
    def _():
        issue(b + 2, jax.lax.rem(b + 2, 4))

    sn = sn_sm[b]
    span_iota = jax.lax.broadcasted_iota(jnp.int32, (1, S), 1)
    valid_span = span_iota < sn
    mask_ref[0] = valid_span.astype(jnp.int32)

    sv = sv_ref[0]
    ev = ev_ref[0]
    lo = sv - ((sv >> 3) << 3)
    cnt = jnp.minimum(ev - sv, _MAX_SPAN_LEN)
    inv = 1.0 / jnp.maximum(cnt, 1).astype(jnp.float32)
    scale = jnp.where(valid_span & (cnt > 0), inv, 0.0)

    k_iota = jax.lax.broadcasted_iota(jnp.int32, (S * _WIN, S), 0)
    off = k_iota - (span_iota * _WIN + lo)
    in_span = off.astype(jnp.uint32) < cnt.astype(jnp.uint32)
    wt = jnp.where(in_span, scale, 0.0)

    wr = pl.multiple_of(wr_sm[b], 8)
    pltpu.make_async_copy(
        emb_hbm.at[b, pl.ds(0, wr), :],
        gbuf.at[slot, pl.ds(0, wr), :],
        sem.at[slot],
    ).wait()

    out_ref[0] = jax.lax.dot_general(
        wt, gbuf[slot], (((0,), (0,)), ((), ())),
        preferred_element_type=jnp.float32)


def kernel(embeddings, span_positions, span_nums, repeated_vector):
    B, L, D = embeddings.shape
    S = span_positions.shape[1]
    sp = span_positions.astype(jnp.int32) + 1
    starts = sp[..., 0]
    ends = sp[..., 1]
    lo = starts & 7
    cnt = jnp.clip(ends - starts, 0, _MAX_SPAN_LEN)
    base_rows = (starts >> 3) * 8
    ntile = (lo + jnp.maximum(cnt, 1) + 7) >> 3
    ntile = jnp.where(jnp.arange(B, dtype=jnp.int32)[:, None] <= 3, 3, ntile)

    m1 = (ntile == 1).astype(jnp.int32)
    m2 = (ntile == 2).astype(jnp.int32)
    m3 = (ntile == 3).astype(jnp.int32)
    c1 = jnp.cumsum(m1, axis=1)
    c2 = jnp.cumsum(m2, axis=1)
    c3 = jnp.cumsum(m3, axis=1)
    n1 = c1[:, -1]
    n2 = c2[:, -1]
    pos = (m1 * (c1 - 1)
           + m2 * (n1[:, None] + c2 - 1)
           + m3 * (n1[:, None] + n2[:, None] + c3 - 1))
    mi_idx = jnp.arange(S, dtype=jnp.int32)[None, :]
    pkv = base_rows | (mi_idx << 16)
    onehot = pos[:, :, None] == mi_idx[None, :, :]
    pk = jnp.sum(jnp.where(onehot, pkv[:, :, None], 0), axis=1).reshape(-1)
    wr = 8 * jnp.sum(ntile, axis=1)

    sn = span_nums.astype(jnp.int32)
    sv = starts.reshape(B, 1, S)
    ev = ends.reshape(B, 1, S)

    body = functools.partial(_pool_body, S=S, D=D)
    grid_spec = pltpu.PrefetchScalarGridSpec(
        num_scalar_prefetch=5,
        grid=(B,),
        in_specs=[pl.BlockSpec(memory_space=pl.ANY),
                  pl.BlockSpec((1, 1, S), lambda b, *_: (b, 0, 0)),
                  pl.BlockSpec((1, 1, S), lambda b, *_: (b, 0, 0))],
        out_specs=[pl.BlockSpec((1, S, D), lambda b, *_: (b, 0, 0)),
                   pl.BlockSpec((1, 1, S), lambda b, *_: (b, 0, 0))],
        scratch_shapes=[
            pltpu.VMEM((4, S * _WIN, D), jnp.float32),
            pltpu.SemaphoreType.DMA((4,)),
        ],
    )
    pooled, maski = pl.pallas_call(
        body,
        grid_spec=grid_spec,
        out_shape=[jax.ShapeDtypeStruct((B, S, D), jnp.float32),
                   jax.ShapeDtypeStruct((B, 1, S), jnp.int32)],
        compiler_params=pltpu.CompilerParams(
            dimension_semantics=("arbitrary",),
        ),
        name="span_mean_pool_dma_mxu",
    )(pk, n1, n2, wr, sn, embeddings, sv, ev)
    return pooled, maski.reshape(B, S) > 0


# speedup vs baseline: 1.4528x; 1.2064x over previous
import functools

import jax
import jax.numpy as jnp
from jax.experimental import pallas as pl
from jax.experimental.pallas import tpu as pltpu

_MAX_SPAN_LEN = 16
_WIN = 24


def _pool_body(pk_sm, nsp_sm, sn_sm, emb_hbm, sv_ref, ev_ref,
               out_ref, mask_ref, gbuf, sem16, sem8, *, S, D):
    b = pl.program_id(0)
    nb = pl.num_programs(0)
    slot = jax.lax.rem(b, 4)

    def issue(bb, sl):
        for mi in range(S):
            v = pk_sm[bb * S + mi]
            base = pl.multiple_of(v & 0xFFFF, 8)
            pltpu.make_async_copy(
                emb_hbm.at[bb, pl.ds(base, 16), :],
                gbuf.at[sl, pl.ds(mi * _WIN, 16), :],
                sem16.at[sl],
            ).start()

            @pl.when((v >> 16) != 0)
            def _():
                pltpu.make_async_copy(
                    emb_hbm.at[bb, pl.ds(base + 16, 8), :],
                    gbuf.at[sl, pl.ds(mi * _WIN + 16, 8), :],
                    sem8.at[sl],
                ).start()

    @pl.when(b == 0)
    def _():
        issue(0, 0)
        issue(1, 1)

    @pl.when(b + 2 < nb)
    def _():
        issue(b + 2, jax.lax.rem(b + 2, 4))

    sn = sn_sm[b]
    span_iota = jax.lax.broadcasted_iota(jnp.int32, (1, S), 1)
    valid_span = span_iota < sn
    mask_ref[0] = valid_span.astype(jnp.int32)

    sv = sv_ref[0]
    ev = ev_ref[0]
    lo = sv - ((sv >> 3) << 3)
    cnt = jnp.minimum(ev - sv, _MAX_SPAN_LEN)
    hi = lo + cnt
    inv = 1.0 / jnp.maximum(cnt, 1).astype(jnp.float32)
    scale = jnp.where(valid_span & (cnt > 0), inv, 0.0)

    k_iota = jax.lax.broadcasted_iota(jnp.int32, (S * _WIN, S), 0)
    mi_iota = jax.lax.broadcasted_iota(jnp.int32, (S * _WIN, S), 1)
    off = k_iota - mi_iota * _WIN
    wt = jnp.where((off >= lo) & (off < hi), scale, 0.0)

    pltpu.make_async_copy(
        emb_hbm.at[b, pl.ds(0, S * 16), :],
        gbuf.at[slot, pl.ds(0, S * 16), :],
        sem16.at[slot],
    ).wait()
    ns = nsp_sm[b]

    @pl.when(ns > 0)
    def _():
        pltpu.make_async_copy(
            emb_hbm.at[b, pl.ds(0, 8 * ns), :],
            gbuf.at[slot, pl.ds(0, 8 * ns), :],
            sem8.at[slot],
        ).wait()

    out_ref[0] = jax.lax.dot_general(
        wt, gbuf[slot], (((0,), (0,)), ((), ())),
        preferred_element_type=jnp.float32)


def kernel(embeddings, span_positions, span_nums, repeated_vector):
    B, L, D = embeddings.shape
    S = span_positions.shape[1]
    sp = span_positions.astype(jnp.int32) + 1
    starts = sp[..., 0]
    ends = sp[..., 1]
    lo = starts & 7
    cnt = jnp.clip(ends - starts, 0, _MAX_SPAN_LEN)
    base_rows = (starts >> 3) * 8
    spill = (lo + jnp.maximum(cnt, 1)) > 16
    spill = spill | (jnp.arange(B, dtype=jnp.int32)[:, None] <= 3)
    pk = (base_rows | (spill.astype(jnp.int32) << 16)).reshape(-1)
    n_spill = spill.astype(jnp.int32).sum(axis=1)
    sn = span_nums.astype(jnp.int32)
    sv = starts.reshape(B, 1, S)
    ev = ends.reshape(B, 1, S)

    body = functools.partial(_pool_body, S=S, D=D)
    grid_spec = pltpu.PrefetchScalarGridSpec(
        num_scalar_prefetch=3,
        grid=(B,),
        in_specs=[pl.BlockSpec(memory_space=pl.ANY),
                  pl.BlockSpec((1, 1, S), lambda b, *_: (b, 0, 0)),
                  pl.BlockSpec((1, 1, S), lambda b, *_: (b, 0, 0))],
        out_specs=[pl.BlockSpec((1, S, D), lambda b, *_: (b, 0, 0)),
                   pl.BlockSpec((1, 1, S), lambda b, *_: (b, 0, 0))],
        scratch_shapes=[
            pltpu.VMEM((4, S * _WIN, D), jnp.float32),
            pltpu.SemaphoreType.DMA((4,)),
            pltpu.SemaphoreType.DMA((4,)),
        ],
    )
    pooled, maski = pl.pallas_call(
        body,
        grid_spec=grid_spec,
        out_shape=[jax.ShapeDtypeStruct((B, S, D), jnp.float32),
                   jax.ShapeDtypeStruct((B, 1, S), jnp.int32)],
        compiler_params=pltpu.CompilerParams(
            dimension_semantics=("arbitrary",),
        ),
        name="span_mean_pool_dma_mxu",
    )(pk, n_spill, sn, embeddings, sv, ev)
    return pooled, maski.reshape(B, S) > 0


# speedup vs baseline: 1.4813x; 1.0196x over previous
import functools

import jax
import jax.numpy as jnp
from jax.experimental import pallas as pl
from jax.experimental.pallas import tpu as pltpu

_MAX_SPAN_LEN = 16
_WIN = 24


def _pool_body(pk_sm, nsp_sm, sn_sm, emb_hbm, sv_ref, ev_ref,
               out_ref, mask_ref, gbuf, sem16, sem8, *, S, D):
    b = pl.program_id(0)
    nb = pl.num_programs(0)
    slot = jax.lax.rem(b, 6)

    def issue(bb, sl):
        for mi in range(S):
            v = pk_sm[bb * S + mi]
            base = pl.multiple_of(v & 0xFFFF, 8)
            pltpu.make_async_copy(
                emb_hbm.at[bb, pl.ds(base, 16), :],
                gbuf.at[sl, pl.ds(mi * _WIN, 16), :],
                sem16.at[sl],
            ).start()

            @pl.when((v >> 16) != 0)
            def _():
                pltpu.make_async_copy(
                    emb_hbm.at[bb, pl.ds(base + 16, 8), :],
                    gbuf.at[sl, pl.ds(mi * _WIN + 16, 8), :],
                    sem8.at[sl],
                ).start()

    @pl.when(b == 0)
    def _():
        issue(0, 0)
        issue(1, 1)
        issue(2, 2)

    @pl.when(b + 3 < nb)
    def _():
        issue(b + 3, jax.lax.rem(b + 3, 6))

    sn = sn_sm[b]
    span_iota = jax.lax.broadcasted_iota(jnp.int32, (1, S), 1)
    valid_span = span_iota < sn
    mask_ref[0] = valid_span.astype(jnp.int32)

    sv = sv_ref[0]
    ev = ev_ref[0]
    lo = sv - ((sv >> 3) << 3)
    cnt = jnp.minimum(ev - sv, _MAX_SPAN_LEN)
    hi = lo + cnt
    inv = 1.0 / jnp.maximum(cnt, 1).astype(jnp.float32)
    scale = jnp.where(valid_span & (cnt > 0), inv, 0.0)

    k_iota = jax.lax.broadcasted_iota(jnp.int32, (S * _WIN, S), 0)
    mi_iota = jax.lax.broadcasted_iota(jnp.int32, (S * _WIN, S), 1)
    off = k_iota - mi_iota * _WIN
    wt = jnp.where((off >= lo) & (off < hi), scale, 0.0)

    pltpu.make_async_copy(
        emb_hbm.at[b, pl.ds(0, S * 16), :],
        gbuf.at[slot, pl.ds(0, S * 16), :],
        sem16.at[slot],
    ).wait()
    ns = nsp_sm[b]

    @pl.when(ns > 0)
    def _():
        pltpu.make_async_copy(
            emb_hbm.at[b, pl.ds(0, 8 * ns), :],
            gbuf.at[slot, pl.ds(0, 8 * ns), :],
            sem8.at[slot],
        ).wait()

    out_ref[0] = jax.lax.dot_general(
        wt, gbuf[slot], (((0,), (0,)), ((), ())),
        preferred_element_type=jnp.float32)


def kernel(embeddings, span_positions, span_nums, repeated_vector):
    B, L, D = embeddings.shape
    S = span_positions.shape[1]
    sp = span_positions.astype(jnp.int32) + 1
    starts = sp[..., 0]
    ends = sp[..., 1]
    lo = starts & 7
    cnt = jnp.clip(ends - starts, 0, _MAX_SPAN_LEN)
    base_rows = (starts >> 3) * 8
    spill = (lo + jnp.maximum(cnt, 1)) > 16
    spill = spill | (jnp.arange(B, dtype=jnp.int32)[:, None] <= 5)
    pk = (base_rows | (spill.astype(jnp.int32) << 16)).reshape(-1)
    n_spill = spill.astype(jnp.int32).sum(axis=1)
    sn = span_nums.astype(jnp.int32)
    sv = starts.reshape(B, 1, S)
    ev = ends.reshape(B, 1, S)

    body = functools.partial(_pool_body, S=S, D=D)
    grid_spec = pltpu.PrefetchScalarGridSpec(
        num_scalar_prefetch=3,
        grid=(B,),
        in_specs=[pl.BlockSpec(memory_space=pl.ANY),
                  pl.BlockSpec((1, 1, S), lambda b, *_: (b, 0, 0)),
                  pl.BlockSpec((1, 1, S), lambda b, *_: (b, 0, 0))],
        out_specs=[pl.BlockSpec((1, S, D), lambda b, *_: (b, 0, 0)),
                   pl.BlockSpec((1, 1, S), lambda b, *_: (b, 0, 0))],
        scratch_shapes=[
            pltpu.VMEM((6, S * _WIN, D), jnp.float32),
            pltpu.SemaphoreType.DMA((6,)),
            pltpu.SemaphoreType.DMA((6,)),
        ],
    )
    pooled, maski = pl.pallas_call(
        body,
        grid_spec=grid_spec,
        out_shape=[jax.ShapeDtypeStruct((B, S, D), jnp.float32),
                   jax.ShapeDtypeStruct((B, 1, S), jnp.int32)],
        compiler_params=pltpu.CompilerParams(
            dimension_semantics=("arbitrary",),
        ),
        name="span_mean_pool_dma_mxu",
    )(pk, n_spill, sn, embeddings, sv, ev)
    return pooled, maski.reshape(B, S) > 0
